# Initial kernel scaffold; baseline (speedup 1.0000x reference)
import functools

import jax
import jax.numpy as jnp
from jax import lax
from jax.experimental import pallas as pl
from jax.experimental.pallas import tpu as pltpu
from jax.experimental.pallas import tpu_sc as plsc

N = 10000
E = 320000
D = 128
H = 128
R = 8

NC = 2
NS = 16
NW = NC * NS
EPT = E // NW
RN = R * N
RN_PAD = 80384
N_PAD = 10240
LPT = RN_PAD // NS
CB1 = 2512
CE1 = 2000
CB3 = 2000
CB5 = 400

_mesh = plsc.VectorSubcoreMesh(core_axis_name="c", subcore_axis_name="s")


@functools.partial(
    pl.kernel,
    out_type=jax.ShapeDtypeStruct((NC, RN_PAD, 16), jnp.float32),
    mesh=_mesh,
    scratch_types=[
        pltpu.VMEM((CE1,), jnp.int32),
        pltpu.VMEM((CB1, 16), jnp.float32),
        pltpu.VMEM_SHARED((RN_PAD, 16), jnp.float32),
    ],
)
def _count_kernel(sidx_hbm, cnt_hbm, idx_v, ones_v, cnt_sh):
    c = lax.axis_index("c")
    s = lax.axis_index("s")
    wid = c * NS + s

    @pl.loop(0, CB1)
    def _zero_fill(i):
        ones_v[i, :] = jnp.zeros((16,), jnp.float32)

    @pl.loop(0, LPT // CB1)
    def _zero_sh(k):
        pltpu.sync_copy(ones_v, cnt_sh.at[pl.ds(s * LPT + k * CB1, CB1)])

    @pl.loop(0, CB1)
    def _one_fill(i):
        ones_v[i, :] = jnp.ones((16,), jnp.float32)

    plsc.subcore_barrier()

    @pl.loop(0, EPT // CE1)
    def _hist(k):
        base = wid * EPT + k * CE1
        pltpu.sync_copy(sidx_hbm.at[pl.ds(base, CE1)], idx_v)
        pltpu.sync_copy(ones_v.at[pl.ds(0, CE1)], cnt_sh.at[idx_v], add=True)

    plsc.subcore_barrier()

    @pl.loop(0, LPT // CB1)
    def _writeout(k):
        pltpu.sync_copy(cnt_sh.at[pl.ds(s * LPT + k * CB1, CB1)],
                        cnt_hbm.at[c, pl.ds(s * LPT + k * CB1, CB1)])


def _inv_body(c_ref, o_ref):
    o_ref[...] = 1.0 / jnp.maximum(c_ref[0] + c_ref[1], 1.0)


def _inv_counts(cnts):
    return pl.pallas_call(
        _inv_body,
        out_shape=jax.ShapeDtypeStruct((RN_PAD // 128, 128), jnp.float32),
    )(cnts)


@functools.partial(
    pl.kernel,
    out_type=jax.ShapeDtypeStruct((E,), jnp.float32),
    mesh=_mesh,
    scratch_types=[
        pltpu.VMEM((RN_PAD,), jnp.float32),
        pltpu.VMEM((CB3,), jnp.int32),
        pltpu.VMEM((CB3,), jnp.float32),
    ],
)
def _scale_kernel(inv_hbm, sidx_hbm, s_hbm, inv_v, sidx_v, s_v):
    c = lax.axis_index("c")
    s = lax.axis_index("s")
    wid = c * NS + s
    pltpu.sync_copy(inv_hbm, inv_v)

    @pl.loop(0, EPT // CB3)
    def _chunk(k):
        base = wid * EPT + k * CB3
        pltpu.sync_copy(sidx_hbm.at[pl.ds(base, CB3)], sidx_v)

        @pl.loop(0, CB3 // 16)
        def _gather(g):
            idx16 = sidx_v[pl.ds(g * 16, 16)]
            s_v[pl.ds(g * 16, 16)] = plsc.load_gather(inv_v, [idx16])

        pltpu.sync_copy(s_v, s_hbm.at[pl.ds(base, CB3)])


MB = 500


def _mm_body_l0(x_ref, w_ref, b_ref, o_ref):
    xb = x_ref[...]
    for r in range(R + 1):
        acc = jnp.dot(xb, w_ref[r], preferred_element_type=jnp.float32)
        if r == R:
            acc = acc + b_ref[...]
        o_ref[r] = acc


def _mm_body_l1(p_ref, w_ref, b_ref, o_ref):
    xb = jnp.maximum(p_ref[0] + p_ref[1], 0.0)
    for r in range(R + 1):
        acc = jnp.dot(xb, w_ref[r], preferred_element_type=jnp.float32)
        if r == R:
            acc = acc + b_ref[...]
        o_ref[r] = acc


def _mm_layer0(x, wall, b):
    return pl.pallas_call(
        _mm_body_l0,
        grid=(N // MB,),
        in_specs=[
            pl.BlockSpec((MB, D), lambda i: (i, 0)),
            pl.BlockSpec((R + 1, D, H), lambda i: (0, 0, 0)),
            pl.BlockSpec((1, H), lambda i: (0, 0)),
        ],
        out_specs=pl.BlockSpec((R + 1, MB, H), lambda i: (0, i, 0)),
        out_shape=jax.ShapeDtypeStruct((R + 1, N, H), jnp.float32),
    )(x, wall, b)


def _mm_layer1(parts, wall, b):
    return pl.pallas_call(
        _mm_body_l1,
        grid=(N // MB,),
        in_specs=[
            pl.BlockSpec((NC, MB, H), lambda i: (0, i, 0)),
            pl.BlockSpec((R + 1, H, H), lambda i: (0, 0, 0)),
            pl.BlockSpec((1, H), lambda i: (0, 0)),
        ],
        out_specs=pl.BlockSpec((R + 1, MB, H), lambda i: (0, i, 0)),
        out_shape=jax.ShapeDtypeStruct((R + 1, N, H), jnp.float32),
    )(parts, wall, b)


@functools.partial(
    pl.kernel,
    out_type=jax.ShapeDtypeStruct((NC, N_PAD, H), jnp.float32),
    mesh=_mesh,
    scratch_types=[
        pltpu.VMEM((CB5,), jnp.int32),
        pltpu.VMEM((CB5,), jnp.int32),
        pltpu.VMEM((CB5,), jnp.float32),
        pltpu.VMEM((CB5, H), jnp.float32),
        pltpu.VMEM_SHARED((N_PAD, H), jnp.float32),
    ],
)
def _agg_kernel(xw_hbm, gidx_hbm, dst_hbm, s_hbm, init_hbm, out_hbm,
                gidx_v, dst_v, s_v, rows_v, acc_sh):
    c = lax.axis_index("c")
    s = lax.axis_index("s")
    wid = c * NS + s
    NR = N_PAD // NS

    pltpu.sync_copy(init_hbm.at[c, pl.ds(s * NR, NR)],
                    acc_sh.at[pl.ds(s * NR, NR)])
    plsc.subcore_barrier()

    @pl.loop(0, EPT // CB5)
    def _chunk(k):
        base = wid * EPT + k * CB5
        pltpu.sync_copy(gidx_hbm.at[pl.ds(base, CB5)], gidx_v)
        pltpu.sync_copy(dst_hbm.at[pl.ds(base, CB5)], dst_v)
        pltpu.sync_copy(s_hbm.at[pl.ds(base, CB5)], s_v)
        pltpu.sync_copy(xw_hbm.at[gidx_v], rows_v)

        @pl.loop(0, CB5)
        def _scale(e):
            spl = plsc.load_gather(s_v, [jnp.full((16,), e, jnp.int32)])
            for hh in range(H // 16):
                sl = pl.ds(hh * 16, 16)
                rows_v[e, sl] = rows_v[e, sl] * spl

        pltpu.sync_copy(rows_v, acc_sh.at[dst_v], add=True)

    plsc.subcore_barrier()
    pltpu.sync_copy(acc_sh.at[pl.ds(s * NR, NR)],
                    out_hbm.at[c, pl.ds(s * NR, NR)])


def _final_body(p_ref, o_ref):
    o_ref[...] = p_ref[0] + p_ref[1]


def _final_add(parts):
    return pl.pallas_call(
        _final_body,
        grid=(N // MB,),
        in_specs=[pl.BlockSpec((NC, MB, H), lambda i: (0, i, 0))],
        out_specs=pl.BlockSpec((MB, H), lambda i: (i, 0)),
        out_shape=jax.ShapeDtypeStruct((N, H), jnp.float32),
    )(parts)


def kernel(x, edge_index, edge_type, W0, root0, b0, W1, root1, b1):
    src = edge_index[0].astype(jnp.int32)
    dst = edge_index[1].astype(jnp.int32)
    et = edge_type.astype(jnp.int32)
    gidx = et * N + src
    sidx = et * N + dst

    cnts = _count_kernel(sidx)
    inv = _inv_counts(cnts[:, :, 0].reshape(NC, RN_PAD // 128, 128))
    svals = _scale_kernel(inv.reshape(RN_PAD), sidx)

    zeros_pad = jnp.zeros((N_PAD, H), jnp.float32)

    wall0 = jnp.concatenate([W0, root0[None]], axis=0)
    xw0 = _mm_layer0(x, wall0, b0.reshape(1, H))
    init0 = jnp.stack(
        [jnp.pad(xw0[R], ((0, N_PAD - N), (0, 0))), zeros_pad])
    parts0 = _agg_kernel(xw0[:R].reshape(RN, H), gidx, dst, svals, init0)

    wall1 = jnp.concatenate([W1, root1[None]], axis=0)
    xw1 = _mm_layer1(parts0[:, :N], wall1, b1.reshape(1, H))
    init1 = jnp.stack(
        [jnp.pad(xw1[R], ((0, N_PAD - N), (0, 0))), zeros_pad])
    parts1 = _agg_kernel(xw1[:R].reshape(RN, H), gidx, dst, svals, init1)

    return _final_add(parts1[:, :N])


# speedup vs baseline: 21.9069x; 21.9069x over previous
import dataclasses
import functools

import jax
import jax.numpy as jnp
from jax import lax
from jax.experimental import pallas as pl
from jax.experimental.pallas import tpu as pltpu
from jax.experimental.pallas import tpu_sc as plsc

N = 10000
E = 320000
D = 128
H = 128
R = 8

NC = 2
NS = 16
NW = NC * NS
EPT = E // NW
RN = R * N
RN_PAD = 82944
N_PAD = 10240
CE1 = 400
CB3 = 2000
CB5 = 400

_mesh = plsc.VectorSubcoreMesh(core_axis_name="c", subcore_axis_name="s")

_sc_params = pltpu.CompilerParams()
if "needs_layout_passes" in pltpu.CompilerParams.__dataclass_fields__:
    _sc_params = dataclasses.replace(_sc_params, needs_layout_passes=False)
if "use_tc_tiling_on_sc" in pltpu.CompilerParams.__dataclass_fields__:
    _sc_params = dataclasses.replace(_sc_params, use_tc_tiling_on_sc=False)


@functools.partial(
    pl.kernel,
    out_type=jax.ShapeDtypeStruct((NC, NS, RN_PAD // 128, 128), jnp.float32),
    mesh=_mesh,
    compiler_params=_sc_params,
    scratch_types=[
        pltpu.VMEM((CE1,), jnp.int32),
        pltpu.VMEM((RN_PAD // 128, 128), jnp.float32),
    ],
)
def _count_kernel(sidx_hbm, cnt_hbm, idx_v, tab_v):
    c = lax.axis_index("c")
    s = lax.axis_index("s")
    wid = c * NS + s
    iota16 = lax.iota(jnp.int32, 16)
    ones16 = jnp.ones((16,), jnp.float32)

    @pl.loop(0, RN_PAD // 128)
    def _zero(i):
        for q in range(8):
            tab_v[i, pl.ds(q * 16, 16)] = jnp.zeros((16,), jnp.float32)

    @pl.loop(0, EPT // CE1)
    def _hist(k):
        base = wid * EPT + k * CE1
        pltpu.sync_copy(sidx_hbm.at[pl.ds(base, CE1)], idx_v)

        @pl.loop(0, CE1 // 16)
        def _grp(g):
            idx16 = idx_v[pl.ds(g * 16, 16)]
            row16 = lax.shift_right_logical(idx16, 7)
            col16 = lax.bitwise_and(idx16, 127)
            for j in range(16):
                plsc.addupdate_scatter(tab_v, [row16, col16], ones16,
                                       mask=iota16 == j)

    pltpu.sync_copy(tab_v, cnt_hbm.at[c, s])


RB = 8


def _inv_body(c_ref, o_ref):
    tot = jnp.sum(c_ref[...], axis=(0, 1))
    o_ref[...] = 1.0 / jnp.maximum(tot, 1.0)


def _inv_counts(cnts):
    return pl.pallas_call(
        _inv_body,
        grid=(RN_PAD // 128 // RB,),
        in_specs=[pl.BlockSpec((NC, NS, RB, 128), lambda i: (0, 0, i, 0))],
        out_specs=pl.BlockSpec((RB, 128), lambda i: (i, 0)),
        out_shape=jax.ShapeDtypeStruct((RN_PAD // 128, 128), jnp.float32),
    )(cnts)


@functools.partial(
    pl.kernel,
    out_type=jax.ShapeDtypeStruct((E,), jnp.float32),
    mesh=_mesh,
    compiler_params=_sc_params,
    scratch_types=[
        pltpu.VMEM((RN_PAD // 128, 128), jnp.float32),
        pltpu.VMEM((CB3,), jnp.int32),
        pltpu.VMEM((CB3,), jnp.float32),
    ],
)
def _scale_kernel(inv_hbm, sidx_hbm, s_hbm, inv_v, sidx_v, s_v):
    c = lax.axis_index("c")
    s = lax.axis_index("s")
    wid = c * NS + s
    pltpu.sync_copy(inv_hbm, inv_v)

    @pl.loop(0, EPT // CB3)
    def _chunk(k):
        base = wid * EPT + k * CB3
        pltpu.sync_copy(sidx_hbm.at[pl.ds(base, CB3)], sidx_v)

        @pl.loop(0, CB3 // 16)
        def _gather(g):
            idx16 = sidx_v[pl.ds(g * 16, 16)]
            row16 = lax.shift_right_logical(idx16, 7)
            col16 = lax.bitwise_and(idx16, 127)
            s_v[pl.ds(g * 16, 16)] = plsc.load_gather(inv_v, [row16, col16])

        pltpu.sync_copy(s_v, s_hbm.at[pl.ds(base, CB3)])


MB = 400


def _mm_body(x_ref, w_ref, b_ref, o_ref):
    xb = x_ref[...]
    for r in range(R + 1):
        acc = jnp.dot(xb, w_ref[r], preferred_element_type=jnp.float32)
        if r == R:
            acc = acc + b_ref[...]
        for h in range(NC):
            o_ref[h, r] = acc[:, h * (H // NC):(h + 1) * (H // NC)]


def _mm_layer(x, wall, b):
    return pl.pallas_call(
        _mm_body,
        grid=(N // MB,),
        in_specs=[
            pl.BlockSpec((MB, D), lambda i: (i, 0)),
            pl.BlockSpec((R + 1, D, H), lambda i: (0, 0, 0)),
            pl.BlockSpec((1, H), lambda i: (0, 0)),
        ],
        out_specs=pl.BlockSpec((NC, R + 1, MB, H // NC),
                               lambda i: (0, 0, i, 0)),
        out_shape=jax.ShapeDtypeStruct((NC, R + 1, N, H // NC), jnp.float32),
    )(x, wall, b)


HC = H // NC
EPC = E // NS


@functools.partial(
    pl.kernel,
    out_type=jax.ShapeDtypeStruct((NC, N_PAD, HC), jnp.float32),
    mesh=_mesh,
    compiler_params=_sc_params,
    scratch_types=[
        pltpu.VMEM((CB5,), jnp.int32),
        pltpu.VMEM((CB5,), jnp.int32),
        pltpu.VMEM((CB5,), jnp.float32),
        pltpu.VMEM((CB5, HC), jnp.float32),
        pltpu.VMEM_SHARED((N_PAD, HC), jnp.float32),
    ],
)
def _agg_kernel(xw_hbm, gidx_hbm, dst_hbm, s_hbm, init_hbm, out_hbm,
                gidx_v, dst_v, s_v, rows_v, acc_sh):
    c = lax.axis_index("c")
    s = lax.axis_index("s")
    NR = N_PAD // NS

    pltpu.sync_copy(init_hbm.at[c, pl.ds(s * NR, NR)],
                    acc_sh.at[pl.ds(s * NR, NR)])
    plsc.subcore_barrier()

    @pl.loop(0, EPC // CB5)
    def _chunk(k):
        base = s * EPC + k * CB5
        pltpu.sync_copy(gidx_hbm.at[pl.ds(base, CB5)], gidx_v)
        pltpu.sync_copy(dst_hbm.at[pl.ds(base, CB5)], dst_v)
        pltpu.sync_copy(s_hbm.at[pl.ds(base, CB5)], s_v)

        @pl.loop(0, CB5 // 16)
        def _off(g):
            sl = pl.ds(g * 16, 16)
            gidx_v[sl] = gidx_v[sl] + c * ((R + 1) * N)

        pltpu.sync_copy(xw_hbm.at[gidx_v], rows_v)

        @pl.loop(0, CB5)
        def _scale(e):
            spl = plsc.load_gather(s_v, [jnp.full((16,), e, jnp.int32)])
            for hh in range(HC // 16):
                sl = pl.ds(hh * 16, 16)
                rows_v[e, sl] = rows_v[e, sl] * spl

        pltpu.sync_copy(rows_v, acc_sh.at[dst_v], add=True)

    plsc.subcore_barrier()
    pltpu.sync_copy(acc_sh.at[pl.ds(s * NR, NR)],
                    out_hbm.at[c, pl.ds(s * NR, NR)])


def _combine_body(p_ref, f_ref, o_ref):
    out = jnp.concatenate([p_ref[0], p_ref[1]], axis=-1)
    o_ref[...] = jnp.where(f_ref[...] > 0.0, jnp.maximum(out, 0.0), out)


def _combine(parts, flag_row):
    return pl.pallas_call(
        _combine_body,
        grid=(N // MB,),
        in_specs=[pl.BlockSpec((NC, MB, HC), lambda i: (0, i, 0)),
                  pl.BlockSpec((1, H), lambda i: (0, 0))],
        out_specs=pl.BlockSpec((MB, H), lambda i: (i, 0)),
        out_shape=jax.ShapeDtypeStruct((N, H), jnp.float32),
    )(parts, flag_row)


def kernel(x, edge_index, edge_type, W0, root0, b0, W1, root1, b1):
    src = edge_index[0].astype(jnp.int32)
    dst = edge_index[1].astype(jnp.int32)
    et = edge_type.astype(jnp.int32)
    gidx = et * N + src
    sidx = et * N + dst

    cnts = _count_kernel(sidx)
    inv = _inv_counts(cnts)
    svals = _scale_kernel(inv, sidx)

    walls = jnp.stack([jnp.concatenate([W0, root0[None]], axis=0),
                       jnp.concatenate([W1, root1[None]], axis=0)])
    biases = jnp.stack([b0.reshape(1, H), b1.reshape(1, H)])
    flags = jnp.stack([jnp.ones((1, H), jnp.float32),
                       jnp.zeros((1, H), jnp.float32)])

    def layer(h, xs):
        wall, b, flag = xs
        xw = _mm_layer(h, wall, b)
        init = jnp.pad(xw[:, R], ((0, 0), (0, N_PAD - N), (0, 0)))
        parts = _agg_kernel(xw.reshape(NC * (R + 1) * N, HC),
                            gidx, dst, svals, init)
        return _combine(parts[:, :N], flag), None

    h_out, _ = lax.scan(layer, x, (walls, biases, flags))
    return h_out
